# baseline (speedup 1.0000x reference)
import functools

import jax
import jax.numpy as jnp
from jax import lax
from jax.experimental import pallas as pl
from jax.experimental.pallas import tpu as pltpu
from jax.experimental.pallas import tpu_sc as plsc

VOCAB = 100000
EMB = 64
U1 = 128
U2 = 512
B = 1024
T = 50
VT = 2048
NVT = (VOCAB + VT - 1) // VT


EP = 128
CHUNK = 80
NCHUNK = 20
HALF = 10


@functools.lru_cache(maxsize=1)
def _make_sc_gather():
    nc, ns = 2, 16
    nw = nc * ns
    tb = T * B
    bpw = tb // nw

    mesh = plsc.VectorSubcoreMesh(core_axis_name="c", subcore_axis_name="s")

    @functools.partial(
        pl.kernel,
        out_type=jax.ShapeDtypeStruct((tb, EP), jnp.float32),
        mesh=mesh,
        scratch_types=[
            pltpu.VMEM((bpw,), jnp.int32),
            pltpu.VMEM((HALF, CHUNK, EP), jnp.float32),
            pltpu.SemaphoreType.DMA,
        ],
    )
    def gather_kernel(emb_hbm, idx_hbm, out_hbm, idx_v, rows_v, sem):
        wid = lax.axis_index("s") * nc + lax.axis_index("c")
        base = wid * bpw
        pltpu.sync_copy(idx_hbm.at[pl.ds(base, bpw)], idx_v)
        for half in range(2):
            copies = []
            for k in range(HALF):
                kk = half * HALF + k
                copies.append(pltpu.async_copy(
                    emb_hbm.at[idx_v.at[pl.ds(kk * CHUNK, CHUNK)]],
                    rows_v.at[k], sem))
            for k, c in enumerate(copies):
                c.wait()
                pltpu.sync_copy(
                    rows_v.at[k],
                    out_hbm.at[pl.ds(base + (half * HALF + k) * CHUNK, CHUNK)])

    return gather_kernel


HB = B // 2


def _lstm_body(e_ref, w1_ref, r1_ref, b1_ref, w2_ref, r2_ref, b2_ref,
               out_ref, h1_ref, c1_ref, h2_ref, c2_ref):
    h1_ref[...] = jnp.zeros((B, U1), jnp.float32)
    c1_ref[...] = jnp.zeros((B, U1), jnp.float32)
    h2_ref[...] = jnp.zeros((B, U2), jnp.float32)
    c2_ref[...] = jnp.zeros((B, U2), jnp.float32)

    def gates(z, units):
        i = jax.nn.sigmoid(z[:, :units])
        f = jax.nn.sigmoid(z[:, units:2 * units])
        g = jnp.tanh(z[:, 2 * units:3 * units])
        o = jax.nn.sigmoid(z[:, 3 * units:])
        return i, f, g, o

    def half_step(t, h):
        sl = pl.ds(h * HB, HB)
        xt = e_ref[pl.ds(t * B + h * HB, HB), :]
        z1 = (jnp.dot(xt, w1_ref[...], preferred_element_type=jnp.float32)
              + jnp.dot(h1_ref[sl, :], r1_ref[...],
                        preferred_element_type=jnp.float32)
              + b1_ref[...])
        i1, f1, g1, o1 = gates(z1, U1)
        c1 = f1 * c1_ref[sl, :] + i1 * g1
        h1 = o1 * jnp.tanh(c1)
        c1_ref[sl, :] = c1
        h1_ref[sl, :] = h1
        z2 = (jnp.dot(h1, w2_ref[...], preferred_element_type=jnp.float32)
              + jnp.dot(h2_ref[sl, :], r2_ref[...],
                        preferred_element_type=jnp.float32)
              + b2_ref[...])
        i2, f2, g2, o2 = gates(z2, U2)
        c2 = f2 * c2_ref[sl, :] + i2 * g2
        h2_ref[sl, :] = o2 * jnp.tanh(c2)
        c2_ref[sl, :] = c2

    def step(t, _):
        half_step(t, 0)
        half_step(t, 1)
        return 0

    lax.fori_loop(0, T, step, 0)
    out_ref[...] = h2_ref[...]


def _lstm(e, W1, R1, b1, W2, R2, b2):
    return pl.pallas_call(
        _lstm_body,
        out_shape=jax.ShapeDtypeStruct((B, U2), jnp.float32),
        scratch_shapes=[
            pltpu.VMEM((B, U1), jnp.float32),
            pltpu.VMEM((B, U1), jnp.float32),
            pltpu.VMEM((B, U2), jnp.float32),
            pltpu.VMEM((B, U2), jnp.float32),
        ],
    )(e, W1, R1, b1.reshape(1, 4 * U1), W2, R2, b2.reshape(1, 4 * U2))


def _logits_tile(h2t_ref, wdt_ref, bd_ref, j):
    l = (jnp.dot(wdt_ref[...], h2t_ref[...],
                 preferred_element_type=jnp.float32)
         + bd_ref[...])
    row = j * VT + lax.broadcasted_iota(jnp.int32, (VT, B), 0)
    return jnp.where(row < VOCAB, l, -jnp.inf)


def _stats_body(h2t_ref, wdt_ref, bd_ref, m_ref, s_ref):
    j = pl.program_id(0)
    l = _logits_tile(h2t_ref, wdt_ref, bd_ref, j)
    bm = jnp.max(l, axis=0, keepdims=True)

    @pl.when(j == 0)
    def _():
        m_ref[...] = bm
        s_ref[...] = jnp.sum(jnp.exp(l - bm), axis=0, keepdims=True)

    @pl.when(j > 0)
    def _():
        m_old = m_ref[...]
        m_new = jnp.maximum(m_old, bm)
        s_ref[...] = (s_ref[...] * jnp.exp(m_old - m_new)
                      + jnp.sum(jnp.exp(l - m_new), axis=0, keepdims=True))
        m_ref[...] = m_new


def _norm_body(h2t_ref, wdt_ref, bd_ref, m_ref, s_ref, out_ref):
    j = pl.program_id(0)
    l = _logits_tile(h2t_ref, wdt_ref, bd_ref, j)
    out_ref[...] = jnp.exp(l - m_ref[...]) / s_ref[...]


def _dense_softmax(h2, Wd, bd):
    wdt = Wd.T
    h2t = h2.T
    bd2 = bd.reshape(VOCAB, 1)
    m, s = pl.pallas_call(
        _stats_body,
        grid=(NVT,),
        in_specs=[
            pl.BlockSpec((U2, B), lambda j: (0, 0)),
            pl.BlockSpec((VT, U2), lambda j: (j, 0)),
            pl.BlockSpec((VT, 1), lambda j: (j, 0)),
        ],
        out_specs=[
            pl.BlockSpec((1, B), lambda j: (0, 0)),
            pl.BlockSpec((1, B), lambda j: (0, 0)),
        ],
        out_shape=[
            jax.ShapeDtypeStruct((1, B), jnp.float32),
            jax.ShapeDtypeStruct((1, B), jnp.float32),
        ],
    )(h2t, wdt, bd2)
    out_t = pl.pallas_call(
        _norm_body,
        grid=(NVT,),
        in_specs=[
            pl.BlockSpec((U2, B), lambda j: (0, 0)),
            pl.BlockSpec((VT, U2), lambda j: (j, 0)),
            pl.BlockSpec((VT, 1), lambda j: (j, 0)),
            pl.BlockSpec((1, B), lambda j: (0, 0)),
            pl.BlockSpec((1, B), lambda j: (0, 0)),
        ],
        out_specs=pl.BlockSpec((VT, B), lambda j: (j, 0)),
        out_shape=jax.ShapeDtypeStruct((VOCAB, B), jnp.float32),
    )(h2t, wdt, bd2, m, s)
    return out_t.T


def kernel(x, emb, W1, R1, b1, W2, R2, b2, Wd, bd):
    idx = x.astype(jnp.int32).T.reshape(T * B)
    emb_p = jnp.pad(emb, ((0, 0), (0, EP - EMB)))
    W1_p = jnp.pad(W1, ((0, EP - EMB), (0, 0)))
    e = _make_sc_gather()(emb_p, idx)
    h2 = _lstm(e, W1_p, R1, b1, W2, R2, b2)
    return _dense_softmax(h2, Wd, bd)


# speedup vs baseline: 1.0204x; 1.0204x over previous
import functools

import jax
import jax.numpy as jnp
from jax import lax
from jax.experimental import pallas as pl
from jax.experimental.pallas import tpu as pltpu
from jax.experimental.pallas import tpu_sc as plsc

VOCAB = 100000
EMB = 64
U1 = 128
U2 = 512
B = 1024
T = 50
VT = 2048
NVT = (VOCAB + VT - 1) // VT


EP = 128
CHUNK = 80
NCHUNK = 20
HALF = 10


@functools.lru_cache(maxsize=1)
def _make_sc_gather():
    nc, ns = 2, 16
    nw = nc * ns
    tb = T * B
    bpw = tb // nw

    mesh = plsc.VectorSubcoreMesh(core_axis_name="c", subcore_axis_name="s")

    @functools.partial(
        pl.kernel,
        out_type=jax.ShapeDtypeStruct((tb, EP), jnp.float32),
        mesh=mesh,
        scratch_types=[
            pltpu.VMEM((bpw,), jnp.int32),
            pltpu.VMEM((HALF, CHUNK, EP), jnp.float32),
            pltpu.SemaphoreType.DMA,
        ],
    )
    def gather_kernel(emb_hbm, idx_hbm, out_hbm, idx_v, rows_v, sem):
        wid = lax.axis_index("s") * nc + lax.axis_index("c")
        base = wid * bpw
        pltpu.sync_copy(idx_hbm.at[pl.ds(base, bpw)], idx_v)
        for half in range(2):
            copies = []
            for k in range(HALF):
                kk = half * HALF + k
                copies.append(pltpu.async_copy(
                    emb_hbm.at[idx_v.at[pl.ds(kk * CHUNK, CHUNK)]],
                    rows_v.at[k], sem))
            for k, c in enumerate(copies):
                c.wait()
                pltpu.sync_copy(
                    rows_v.at[k],
                    out_hbm.at[pl.ds(base + (half * HALF + k) * CHUNK, CHUNK)])

    return gather_kernel


HB = B // 2


def _lstm_body(e_ref, w1_ref, r1_ref, b1_ref, w2_ref, r2_ref, b2_ref,
               out_ref, h1a_ref, c1a_ref, h2a_ref, c2a_ref,
               h1b_ref, c1b_ref, h2b_ref, c2b_ref):
    for r in (h1a_ref, c1a_ref, h1b_ref, c1b_ref):
        r[...] = jnp.zeros((HB, U1), jnp.float32)
    for r in (h2a_ref, c2a_ref, h2b_ref, c2b_ref):
        r[...] = jnp.zeros((HB, U2), jnp.float32)

    def gates(z, units):
        i = jax.nn.sigmoid(z[:, :units])
        f = jax.nn.sigmoid(z[:, units:2 * units])
        g = jnp.tanh(z[:, 2 * units:3 * units])
        o = jax.nn.sigmoid(z[:, 3 * units:])
        return i, f, g, o

    def half_step(t, h, h1_ref, c1_ref, h2_ref, c2_ref):
        xt = e_ref[pl.ds(t * B + h * HB, HB), :]
        z1 = (jnp.dot(xt, w1_ref[...], preferred_element_type=jnp.float32)
              + jnp.dot(h1_ref[...], r1_ref[...],
                        preferred_element_type=jnp.float32)
              + b1_ref[...])
        i1, f1, g1, o1 = gates(z1, U1)
        c1 = f1 * c1_ref[...] + i1 * g1
        h1 = o1 * jnp.tanh(c1)
        c1_ref[...] = c1
        h1_ref[...] = h1
        z2 = (jnp.dot(h1, w2_ref[...], preferred_element_type=jnp.float32)
              + jnp.dot(h2_ref[...], r2_ref[...],
                        preferred_element_type=jnp.float32)
              + b2_ref[...])
        i2, f2, g2, o2 = gates(z2, U2)
        c2 = f2 * c2_ref[...] + i2 * g2
        h2_ref[...] = o2 * jnp.tanh(c2)
        c2_ref[...] = c2

    def step(t, _):
        half_step(t, 0, h1a_ref, c1a_ref, h2a_ref, c2a_ref)
        half_step(t, 1, h1b_ref, c1b_ref, h2b_ref, c2b_ref)
        return 0

    lax.fori_loop(0, T, step, 0)
    out_ref[pl.ds(0, HB), :] = h2a_ref[...]
    out_ref[pl.ds(HB, HB), :] = h2b_ref[...]


def _lstm(e, W1, R1, b1, W2, R2, b2):
    return pl.pallas_call(
        _lstm_body,
        out_shape=jax.ShapeDtypeStruct((B, U2), jnp.float32),
        scratch_shapes=[
            pltpu.VMEM((HB, U1), jnp.float32),
            pltpu.VMEM((HB, U1), jnp.float32),
            pltpu.VMEM((HB, U2), jnp.float32),
            pltpu.VMEM((HB, U2), jnp.float32),
            pltpu.VMEM((HB, U1), jnp.float32),
            pltpu.VMEM((HB, U1), jnp.float32),
            pltpu.VMEM((HB, U2), jnp.float32),
            pltpu.VMEM((HB, U2), jnp.float32),
        ],
    )(e, W1, R1, b1.reshape(1, 4 * U1), W2, R2, b2.reshape(1, 4 * U2))


HVT = VT // 2


def _logits_sub(h2t_ref, wdt_ref, bd_ref, j, k):
    sl = pl.ds(k * HVT, HVT)
    l = (jnp.dot(wdt_ref[sl, :], h2t_ref[...],
                 preferred_element_type=jnp.float32)
         + bd_ref[sl, :])
    row = j * VT + k * HVT + lax.broadcasted_iota(jnp.int32, (HVT, B), 0)
    return jnp.where(row < VOCAB, l, -jnp.inf)


def _stats_body(h2t_ref, wdt_ref, bd_ref, m_ref, s_ref):
    j = pl.program_id(0)
    l0 = _logits_sub(h2t_ref, wdt_ref, bd_ref, j, 0)
    bm0 = jnp.max(l0, axis=0, keepdims=True)
    e0r = jnp.sum(jnp.exp(l0 - bm0), axis=0, keepdims=True)
    l1 = _logits_sub(h2t_ref, wdt_ref, bd_ref, j, 1)
    bm1 = jnp.max(l1, axis=0, keepdims=True)
    e1r = jnp.sum(jnp.exp(l1 - bm1), axis=0, keepdims=True)
    bm = jnp.maximum(bm0, bm1)
    e0 = e0r * jnp.exp(bm0 - bm)
    e1 = e1r * jnp.exp(bm1 - bm)

    @pl.when(j == 0)
    def _():
        m_ref[...] = bm
        s_ref[...] = e0 + e1

    @pl.when(j > 0)
    def _():
        m_old = m_ref[...]
        m_new = jnp.maximum(m_old, bm)
        scale_old = jnp.exp(m_old - m_new)
        scale_new = jnp.exp(bm - m_new)
        s_ref[...] = s_ref[...] * scale_old + (e0 + e1) * scale_new
        m_ref[...] = m_new


def _norm_body(h2t_ref, wdt_ref, bd_ref, m_ref, s_ref, out_ref):
    j = pl.program_id(0)
    rs = 1.0 / s_ref[...]
    m = m_ref[...]
    l0 = _logits_sub(h2t_ref, wdt_ref, bd_ref, j, 0)
    l1 = _logits_sub(h2t_ref, wdt_ref, bd_ref, j, 1)
    out_ref[pl.ds(0, HVT), :] = jnp.exp(l0 - m) * rs
    out_ref[pl.ds(HVT, HVT), :] = jnp.exp(l1 - m) * rs


def _dense_softmax(h2, Wd, bd):
    wdt = Wd.T
    h2t = h2.T
    bd2 = bd.reshape(VOCAB, 1)
    m, s = pl.pallas_call(
        _stats_body,
        grid=(NVT,),
        in_specs=[
            pl.BlockSpec((U2, B), lambda j: (0, 0)),
            pl.BlockSpec((VT, U2), lambda j: (j, 0)),
            pl.BlockSpec((VT, 1), lambda j: (j, 0)),
        ],
        out_specs=[
            pl.BlockSpec((1, B), lambda j: (0, 0)),
            pl.BlockSpec((1, B), lambda j: (0, 0)),
        ],
        out_shape=[
            jax.ShapeDtypeStruct((1, B), jnp.float32),
            jax.ShapeDtypeStruct((1, B), jnp.float32),
        ],
    )(h2t, wdt, bd2)
    out_t = pl.pallas_call(
        _norm_body,
        grid=(NVT,),
        in_specs=[
            pl.BlockSpec((U2, B), lambda j: (0, 0)),
            pl.BlockSpec((VT, U2), lambda j: (j, 0)),
            pl.BlockSpec((VT, 1), lambda j: (j, 0)),
            pl.BlockSpec((1, B), lambda j: (0, 0)),
            pl.BlockSpec((1, B), lambda j: (0, 0)),
        ],
        out_specs=pl.BlockSpec((VT, B), lambda j: (j, 0)),
        out_shape=jax.ShapeDtypeStruct((VOCAB, B), jnp.float32),
    )(h2t, wdt, bd2, m, s)
    return out_t.T


def kernel(x, emb, W1, R1, b1, W2, R2, b2, Wd, bd):
    idx = x.astype(jnp.int32).T.reshape(T * B)
    emb_p = jnp.pad(emb, ((0, 0), (0, EP - EMB)))
    W1_p = jnp.pad(W1, ((0, EP - EMB), (0, 0)))
    e = _make_sc_gather()(emb_p, idx)
    h2 = _lstm(e, W1_p, R1, b1, W2, R2, b2)
    return _dense_softmax(h2, Wd, bd)


# speedup vs baseline: 1.0450x; 1.0241x over previous
import functools

import jax
import jax.numpy as jnp
from jax import lax
from jax.experimental import pallas as pl
from jax.experimental.pallas import tpu as pltpu
from jax.experimental.pallas import tpu_sc as plsc

VOCAB = 100000
EMB = 64
U1 = 128
U2 = 512
B = 1024
T = 50
VT = 2048
NVT = (VOCAB + VT - 1) // VT


EP = 128
CHUNK = 80
NCHUNK = 20
HALF = 10


@functools.lru_cache(maxsize=1)
def _make_sc_gather():
    nc, ns = 2, 16
    nw = nc * ns
    tb = T * B
    bpw = tb // nw

    mesh = plsc.VectorSubcoreMesh(core_axis_name="c", subcore_axis_name="s")

    @functools.partial(
        pl.kernel,
        out_type=jax.ShapeDtypeStruct((tb, EMB), jnp.float32),
        mesh=mesh,
        compiler_params=pltpu.CompilerParams(use_tc_tiling_on_sc=False),
        scratch_types=[
            pltpu.VMEM((bpw,), jnp.int32),
            pltpu.VMEM((HALF, CHUNK, EMB), jnp.float32),
            pltpu.SemaphoreType.DMA,
        ],
    )
    def gather_kernel(emb_hbm, idx_hbm, out_hbm, idx_v, rows_v, sem):
        wid = lax.axis_index("s") * nc + lax.axis_index("c")
        base = wid * bpw
        pltpu.sync_copy(idx_hbm.at[pl.ds(base, bpw)], idx_v)
        for half in range(2):
            copies = []
            for k in range(HALF):
                kk = half * HALF + k
                copies.append(pltpu.async_copy(
                    emb_hbm.at[idx_v.at[pl.ds(kk * CHUNK, CHUNK)]],
                    rows_v.at[k], sem))
            for k, c in enumerate(copies):
                c.wait()
                pltpu.sync_copy(
                    rows_v.at[k],
                    out_hbm.at[pl.ds(base + (half * HALF + k) * CHUNK, CHUNK)])

    return gather_kernel


HB = B // 2


def _lstm_body(e_ref, w1_ref, r1_ref, b1_ref, w2_ref, r2_ref, b2_ref,
               out_ref, h1a_ref, c1a_ref, h2a_ref, c2a_ref,
               h1b_ref, c1b_ref, h2b_ref, c2b_ref):
    for r in (h1a_ref, c1a_ref, h1b_ref, c1b_ref):
        r[...] = jnp.zeros((HB, U1), jnp.float32)
    for r in (h2a_ref, c2a_ref, h2b_ref, c2b_ref):
        r[...] = jnp.zeros((HB, U2), jnp.float32)

    def gates(z, units):
        i = jax.nn.sigmoid(z[:, :units])
        f = jax.nn.sigmoid(z[:, units:2 * units])
        g = jnp.tanh(z[:, 2 * units:3 * units])
        o = jax.nn.sigmoid(z[:, 3 * units:])
        return i, f, g, o

    def half_step(t, h, h1_ref, c1_ref, h2_ref, c2_ref):
        xt = e_ref[pl.ds(t * B + h * HB, HB), :]
        z1 = (jnp.dot(xt, w1_ref[...], preferred_element_type=jnp.float32)
              + jnp.dot(h1_ref[...], r1_ref[...],
                        preferred_element_type=jnp.float32)
              + b1_ref[...])
        i1, f1, g1, o1 = gates(z1, U1)
        c1 = f1 * c1_ref[...] + i1 * g1
        h1 = o1 * jnp.tanh(c1)
        c1_ref[...] = c1
        h1_ref[...] = h1
        z2 = (jnp.dot(h1, w2_ref[...], preferred_element_type=jnp.float32)
              + jnp.dot(h2_ref[...], r2_ref[...],
                        preferred_element_type=jnp.float32)
              + b2_ref[...])
        i2, f2, g2, o2 = gates(z2, U2)
        c2 = f2 * c2_ref[...] + i2 * g2
        h2_ref[...] = o2 * jnp.tanh(c2)
        c2_ref[...] = c2

    def step(t, _):
        half_step(t, 0, h1a_ref, c1a_ref, h2a_ref, c2a_ref)
        half_step(t, 1, h1b_ref, c1b_ref, h2b_ref, c2b_ref)
        return 0

    lax.fori_loop(0, T, step, 0)
    out_ref[pl.ds(0, HB), :] = h2a_ref[...]
    out_ref[pl.ds(HB, HB), :] = h2b_ref[...]


def _lstm(e, W1, R1, b1, W2, R2, b2):
    return pl.pallas_call(
        _lstm_body,
        out_shape=jax.ShapeDtypeStruct((B, U2), jnp.float32),
        scratch_shapes=[
            pltpu.VMEM((HB, U1), jnp.float32),
            pltpu.VMEM((HB, U1), jnp.float32),
            pltpu.VMEM((HB, U2), jnp.float32),
            pltpu.VMEM((HB, U2), jnp.float32),
            pltpu.VMEM((HB, U1), jnp.float32),
            pltpu.VMEM((HB, U1), jnp.float32),
            pltpu.VMEM((HB, U2), jnp.float32),
            pltpu.VMEM((HB, U2), jnp.float32),
        ],
    )(e, W1, R1, b1.reshape(1, 4 * U1), W2, R2, b2.reshape(1, 4 * U2))


HVT = VT // 2


def _logits_sub(h2t_ref, wdt_ref, bd_ref, j, k):
    sl = pl.ds(k * HVT, HVT)
    bd_col = bd_ref[sl].reshape(HVT, 1)
    l = (jnp.dot(wdt_ref[sl, :], h2t_ref[...],
                 preferred_element_type=jnp.float32)
         + bd_col)
    row = j * VT + k * HVT + lax.broadcasted_iota(jnp.int32, (HVT, B), 0)
    return jnp.where(row < VOCAB, l, -jnp.inf)


def _stats_body(h2t_ref, wdt_ref, bd_ref, m_ref, s_ref):
    j = pl.program_id(0)
    l0 = _logits_sub(h2t_ref, wdt_ref, bd_ref, j, 0)
    bm0 = jnp.max(l0, axis=0, keepdims=True)
    e0r = jnp.sum(jnp.exp(l0 - bm0), axis=0, keepdims=True)
    l1 = _logits_sub(h2t_ref, wdt_ref, bd_ref, j, 1)
    bm1 = jnp.max(l1, axis=0, keepdims=True)
    e1r = jnp.sum(jnp.exp(l1 - bm1), axis=0, keepdims=True)
    bm = jnp.maximum(bm0, bm1)
    e0 = e0r * jnp.exp(bm0 - bm)
    e1 = e1r * jnp.exp(bm1 - bm)

    @pl.when(j == 0)
    def _():
        m_ref[...] = bm
        s_ref[...] = e0 + e1

    @pl.when(j > 0)
    def _():
        m_old = m_ref[...]
        m_new = jnp.maximum(m_old, bm)
        scale_old = jnp.exp(m_old - m_new)
        scale_new = jnp.exp(bm - m_new)
        s_ref[...] = s_ref[...] * scale_old + (e0 + e1) * scale_new
        m_ref[...] = m_new


def _norm_body(h2t_ref, wdt_ref, bd_ref, m_ref, s_ref, out_ref):
    j = pl.program_id(0)
    rs = 1.0 / s_ref[...]
    m = m_ref[...]
    l0 = _logits_sub(h2t_ref, wdt_ref, bd_ref, j, 0)
    l1 = _logits_sub(h2t_ref, wdt_ref, bd_ref, j, 1)
    out_ref[pl.ds(0, HVT), :] = jnp.exp(l0 - m) * rs
    out_ref[pl.ds(HVT, HVT), :] = jnp.exp(l1 - m) * rs


def _dense_softmax(h2, Wd, bd):
    wdt = Wd.T
    h2t = h2.T
    bd2 = bd
    m, s = pl.pallas_call(
        _stats_body,
        grid=(NVT,),
        in_specs=[
            pl.BlockSpec((U2, B), lambda j: (0, 0)),
            pl.BlockSpec((VT, U2), lambda j: (j, 0)),
            pl.BlockSpec((VT,), lambda j: (j,)),
        ],
        out_specs=[
            pl.BlockSpec((1, B), lambda j: (0, 0)),
            pl.BlockSpec((1, B), lambda j: (0, 0)),
        ],
        out_shape=[
            jax.ShapeDtypeStruct((1, B), jnp.float32),
            jax.ShapeDtypeStruct((1, B), jnp.float32),
        ],
    )(h2t, wdt, bd2)
    out_t = pl.pallas_call(
        _norm_body,
        grid=(NVT,),
        in_specs=[
            pl.BlockSpec((U2, B), lambda j: (0, 0)),
            pl.BlockSpec((VT, U2), lambda j: (j, 0)),
            pl.BlockSpec((VT,), lambda j: (j,)),
            pl.BlockSpec((1, B), lambda j: (0, 0)),
            pl.BlockSpec((1, B), lambda j: (0, 0)),
        ],
        out_specs=pl.BlockSpec((VT, B), lambda j: (j, 0)),
        out_shape=jax.ShapeDtypeStruct((VOCAB, B), jnp.float32),
    )(h2t, wdt, bd2, m, s)
    return out_t.T


def kernel(x, emb, W1, R1, b1, W2, R2, b2, Wd, bd):
    idx = x.astype(jnp.int32).T.reshape(T * B)
    e = _make_sc_gather()(emb, idx)
    h2 = _lstm(e, W1, R1, b1, W2, R2, b2)
    return _dense_softmax(h2, Wd, bd)
